# Initial kernel scaffold; baseline (speedup 1.0000x reference)
import jax
import jax.numpy as jnp
from jax.experimental import pallas as pl


def kernel(seqs, adjs, comp, weight, bias):
    raise NotImplementedError("write your pallas kernel here")


# speedup vs baseline: 1.5478x; 1.5478x over previous
import jax
import jax.numpy as jnp
from jax.experimental import pallas as pl
from jax.experimental.pallas import tpu as pltpu

_N = 10000
_FT = 128
_EPS = 1e-15
_MAXNORM = 1.0 - 1e-3


def _artanh(x):
    x = jnp.clip(x, -1.0 + 1e-5, 1.0 - 1e-5)
    return 0.5 * jnp.log((1.0 + x) / (1.0 - x))


def _rownorm(x):
    return jnp.maximum(jnp.sqrt(jnp.sum(x * x, axis=-1, keepdims=True)), _EPS)


def _project(x):
    n = _rownorm(x)
    return jnp.where(n > _MAXNORM, x * (_MAXNORM / n), x)


def _expmap0(u):
    n = _rownorm(u)
    return jnp.tanh(n) * u / n


def _logmap0(y):
    n = _rownorm(y)
    return _artanh(n) * y / n


def _tangent_kernel(seq_ref, w_ref, bias_ref, xt_ref):
    x = seq_ref[...]
    h = _expmap0(x)
    xn = _rownorm(h)
    mx = jax.lax.dot_general(h, w_ref[...], (((1,), (1,)), ((), ())),
                             preferred_element_type=jnp.float32)
    mxn = _rownorm(mx)
    res = jnp.tanh(mxn / xn * _artanh(xn)) * mx / mxn
    res = jnp.where(mxn <= 1e-10, jnp.zeros_like(res), res)
    h = _project(res)
    hb = _project(_expmap0(bias_ref[...]))
    x2 = jnp.sum(h * h, axis=-1, keepdims=True)
    y2 = jnp.sum(hb * hb, axis=-1, keepdims=True)
    xy = jnp.sum(h * hb, axis=-1, keepdims=True)
    num = (1.0 + 2.0 * xy + y2) * h + (1.0 - x2) * hb
    den = 1.0 + 2.0 * xy + x2 * y2
    h = _project(num / jnp.maximum(den, _EPS))
    xt_ref[...] = _logmap0(h)


def _agg_kernel(adj_ref, xt_ref, out_ref):
    s = jnp.dot(adj_ref[...], xt_ref[...], preferred_element_type=jnp.float32)
    h = _project(_expmap0(s))
    ht = jnp.maximum(_logmap0(h), 0.0)
    h = _project(_expmap0(ht))
    out_ref[...] = _logmap0(h)


def kernel(seqs, adjs, comp, weight, bias):
    w = (comp @ weight.reshape(weight.shape[0], -1)).reshape(1, _FT, _FT)[0]
    seq = seqs[0]
    adj = adjs[0]

    b1 = 2000
    xt = pl.pallas_call(
        _tangent_kernel,
        grid=(_N // b1,),
        in_specs=[
            pl.BlockSpec((b1, _FT), lambda i: (i, 0)),
            pl.BlockSpec((_FT, _FT), lambda i: (0, 0)),
            pl.BlockSpec((1, _FT), lambda i: (0, 0)),
        ],
        out_specs=pl.BlockSpec((b1, _FT), lambda i: (i, 0)),
        out_shape=jax.ShapeDtypeStruct((_N, _FT), jnp.float32),
        compiler_params=pltpu.CompilerParams(
            dimension_semantics=("parallel",)),
    )(seq, w, bias)

    b2 = 200
    out = pl.pallas_call(
        _agg_kernel,
        grid=(_N // b2,),
        in_specs=[
            pl.BlockSpec((b2, _N), lambda i: (i, 0)),
            pl.BlockSpec((_N, _FT), lambda i: (0, 0)),
        ],
        out_specs=pl.BlockSpec((b2, _FT), lambda i: (i, 0)),
        out_shape=jax.ShapeDtypeStruct((_N, _FT), jnp.float32),
        compiler_params=pltpu.CompilerParams(
            dimension_semantics=("parallel",)),
    )(adj, xt)
    return out


# speedup vs baseline: 1.6825x; 1.0870x over previous
import jax
import jax.numpy as jnp
from jax.experimental import pallas as pl
from jax.experimental.pallas import tpu as pltpu

_N = 10000
_FT = 128
_EPS = 1e-15
_MAXNORM = 1.0 - 1e-3


def _artanh(x):
    x = jnp.clip(x, -1.0 + 1e-5, 1.0 - 1e-5)
    return 0.5 * jnp.log((1.0 + x) / (1.0 - x))


def _rownorm(x):
    return jnp.maximum(jnp.sqrt(jnp.sum(x * x, axis=-1, keepdims=True)), _EPS)


def _project(x):
    n = _rownorm(x)
    return jnp.where(n > _MAXNORM, x * (_MAXNORM / n), x)


def _expmap0(u):
    n = _rownorm(u)
    return jnp.tanh(n) * u / n


def _logmap0(y):
    n = _rownorm(y)
    return _artanh(n) * y / n


def _tangent_kernel(seq_ref, w_ref, bias_ref, xt_ref):
    u = seq_ref[...]
    un = _rownorm(u)
    p = jax.lax.dot_general(u, w_ref[...], (((1,), (1,)), ((), ())),
                            preferred_element_type=jnp.float32)
    pn = _rownorm(p)
    res = jnp.tanh(pn * _artanh(jnp.tanh(un)) / un) * p / pn
    h = _project(res)
    hb = _project(_expmap0(bias_ref[...]))
    x2 = jnp.sum(h * h, axis=-1, keepdims=True)
    y2 = jnp.sum(hb * hb, axis=-1, keepdims=True)
    xy = jnp.sum(h * hb, axis=-1, keepdims=True)
    num = (1.0 + 2.0 * xy + y2) * h + (1.0 - x2) * hb
    den = 1.0 + 2.0 * xy + x2 * y2
    h = _project(num / jnp.maximum(den, _EPS))
    xt_ref[...] = _logmap0(h)


def _agg_kernel(adj_ref, xt_ref, out_ref):
    s = jnp.dot(adj_ref[...], xt_ref[...], preferred_element_type=jnp.float32)
    h = _project(_expmap0(s))
    ht = jnp.maximum(_logmap0(h), 0.0)
    h = _project(_expmap0(ht))
    out_ref[...] = _logmap0(h)


def kernel(seqs, adjs, comp, weight, bias):
    w = (comp @ weight.reshape(weight.shape[0], -1)).reshape(1, _FT, _FT)[0]
    seq = seqs[0]
    adj = adjs[0]

    b1 = 2000
    xt = pl.pallas_call(
        _tangent_kernel,
        grid=(_N // b1,),
        in_specs=[
            pl.BlockSpec((b1, _FT), lambda i: (i, 0)),
            pl.BlockSpec((_FT, _FT), lambda i: (0, 0)),
            pl.BlockSpec((1, _FT), lambda i: (0, 0)),
        ],
        out_specs=pl.BlockSpec((b1, _FT), lambda i: (i, 0)),
        out_shape=jax.ShapeDtypeStruct((_N, _FT), jnp.float32),
        compiler_params=pltpu.CompilerParams(
            dimension_semantics=("parallel",)),
    )(seq, w, bias)

    b2 = 400
    out = pl.pallas_call(
        _agg_kernel,
        grid=(_N // b2,),
        in_specs=[
            pl.BlockSpec((b2, _N), lambda i: (i, 0)),
            pl.BlockSpec((_N, _FT), lambda i: (0, 0)),
        ],
        out_specs=pl.BlockSpec((b2, _FT), lambda i: (i, 0)),
        out_shape=jax.ShapeDtypeStruct((_N, _FT), jnp.float32),
        compiler_params=pltpu.CompilerParams(
            dimension_semantics=("parallel",),
            vmem_limit_bytes=100 * 1024 * 1024),
    )(adj, xt)
    return out
